# baseline (speedup 1.0000x reference)
import functools

import jax
import jax.numpy as jnp
from jax.experimental import pallas as pl
from jax.experimental.pallas import tpu as pltpu

NUM_EXPERTS = 64
HIDDEN = 768
FF = 512
TOKENS = 2048
BLOCK = 128
GRID = TOKENS // BLOCK + NUM_EXPERTS
DUMP = TOKENS


def _moe_kernel(nblk_ref, bexp_ref, rid_ref,
                w_ref, x_ref, gu_w_ref, dn_w_ref,
                out_ref,
                x_s, y_s):
    b = pl.program_id(0)

    @pl.when(b < nblk_ref[0])
    def _():
        base = b * BLOCK

        def gather_row(r, _):
            t = rid_ref[base + r]
            g = jnp.minimum(t, TOKENS - 1)
            x_s[pl.ds(r, 1), :] = x_ref[pl.ds(g, 1), :]
            return 0

        jax.lax.fori_loop(0, BLOCK, gather_row, 0, unroll=8)

        x = x_s[...].astype(jnp.bfloat16)
        gu = jax.lax.dot_general(
            x, gu_w_ref[0], (((1,), (1,)), ((), ())),
            preferred_element_type=jnp.float32)
        gate = gu[:, :FF]
        up = gu[:, FF:]
        h = (gate * jax.nn.sigmoid(gate) * up).astype(jnp.bfloat16)
        y = jax.lax.dot_general(
            h, dn_w_ref[0], (((1,), (1,)), ((), ())),
            preferred_element_type=jnp.float32)
        y_s[...] = y * w_ref[0, 0, :][:, None]

        def scatter_row(r, _):
            t = rid_ref[base + r]
            out_ref[pl.ds(t, 1), :] = y_s[pl.ds(r, 1), :]
            return 0

        jax.lax.fori_loop(0, BLOCK, scatter_row, 0, unroll=8)


@jax.jit
def kernel(hidden_states, top_k_index, top_k_weights, gate_up_proj, down_proj):
    e = top_k_index[:, 0].astype(jnp.int32)
    sort_idx = jnp.argsort(e).astype(jnp.int32)
    counts = jnp.bincount(e, length=NUM_EXPERTS).astype(jnp.int32)
    nb = (counts + BLOCK - 1) // BLOCK
    start_blk = jnp.cumsum(nb) - nb
    nblocks = jnp.sum(nb).astype(jnp.int32)
    offsets = jnp.cumsum(counts) - counts

    b_ids = jnp.arange(GRID, dtype=jnp.int32)
    e_of_b = (jnp.searchsorted(start_blk, b_ids, side="right") - 1).astype(
        jnp.int32)
    p = (b_ids - start_blk[e_of_b])[:, None] * BLOCK + jnp.arange(
        BLOCK, dtype=jnp.int32)[None, :]
    valid = p < counts[e_of_b][:, None]
    spos = jnp.clip(offsets[e_of_b][:, None] + p, 0, TOKENS - 1)
    tok = jnp.where(valid, sort_idx[spos], DUMP).astype(jnp.int32)
    wts = jnp.where(valid, top_k_weights[jnp.clip(tok, 0, TOKENS - 1), 0], 0.0)
    wts = wts.astype(jnp.float32).reshape(GRID, 1, BLOCK)

    grid_spec = pltpu.PrefetchScalarGridSpec(
        num_scalar_prefetch=3,
        grid=(GRID,),
        in_specs=[
            pl.BlockSpec((1, 1, BLOCK), lambda b, n, be, rid: (b, 0, 0)),
            pl.BlockSpec((TOKENS, HIDDEN), lambda b, n, be, rid: (0, 0)),
            pl.BlockSpec((1, 2 * FF, HIDDEN),
                         lambda b, n, be, rid: (be[b], 0, 0)),
            pl.BlockSpec((1, HIDDEN, FF),
                         lambda b, n, be, rid: (be[b], 0, 0)),
        ],
        out_specs=pl.BlockSpec((TOKENS + 8, HIDDEN),
                               lambda b, n, be, rid: (0, 0)),
        scratch_shapes=[
            pltpu.VMEM((BLOCK, HIDDEN), jnp.float32),
            pltpu.VMEM((BLOCK, HIDDEN), jnp.float32),
        ],
    )

    out = pl.pallas_call(
        _moe_kernel,
        grid_spec=grid_spec,
        out_shape=jax.ShapeDtypeStruct((TOKENS + 8, HIDDEN), jnp.float32),
    )(nblocks.reshape(1), e_of_b, tok.reshape(-1),
      wts, hidden_states,
      gate_up_proj.astype(jnp.bfloat16), down_proj.astype(jnp.bfloat16))
    return out[:TOKENS]


# speedup vs baseline: 1.3704x; 1.3704x over previous
import functools

import jax
import jax.numpy as jnp
from jax.experimental import pallas as pl
from jax.experimental.pallas import tpu as pltpu

NUM_EXPERTS = 64
HIDDEN = 768
FF = 512
TOKENS = 2048
BLOCK = 128
GRID = TOKENS // BLOCK + NUM_EXPERTS
DUMP = TOKENS


def _moe_kernel(nblk_ref, bexp_ref, rid_ref,
                w_ref, x_ref, gu_w_ref, dn_w_ref,
                out_ref,
                x_s, y_s):
    b = pl.program_id(0)

    @pl.when(b < nblk_ref[0])
    def _():
        base = b * BLOCK

        def gather_row(r, _):
            t = rid_ref[base + r]
            g = jnp.minimum(t, TOKENS - 1)
            x_s[pl.ds(r, 1), :] = x_ref[pl.ds(g, 1), :]
            return 0

        jax.lax.fori_loop(0, BLOCK, gather_row, 0, unroll=8)

        x = x_s[...]
        gu = jax.lax.dot_general(
            x, gu_w_ref[0], (((1,), (1,)), ((), ())),
            preferred_element_type=jnp.float32)
        gate = gu[:, :FF]
        up = gu[:, FF:]
        h = gate * jax.nn.sigmoid(gate) * up
        y = jax.lax.dot_general(
            h, dn_w_ref[0], (((1,), (1,)), ((), ())),
            preferred_element_type=jnp.float32)
        y_s[...] = y * w_ref[0, 0, :][:, None]

        def scatter_row(r, _):
            t = rid_ref[base + r]
            out_ref[pl.ds(t, 1), :] = y_s[pl.ds(r, 1), :]
            return 0

        jax.lax.fori_loop(0, BLOCK, scatter_row, 0, unroll=8)


@jax.jit
def kernel(hidden_states, top_k_index, top_k_weights, gate_up_proj, down_proj):
    e = top_k_index[:, 0].astype(jnp.int32)
    sort_idx = jnp.argsort(e).astype(jnp.int32)
    counts = jnp.bincount(e, length=NUM_EXPERTS).astype(jnp.int32)
    nb = (counts + BLOCK - 1) // BLOCK
    start_blk = jnp.cumsum(nb) - nb
    nblocks = jnp.sum(nb).astype(jnp.int32)
    offsets = jnp.cumsum(counts) - counts

    b_ids = jnp.arange(GRID, dtype=jnp.int32)
    e_of_b = (jnp.searchsorted(start_blk, b_ids, side="right") - 1).astype(
        jnp.int32)
    p = (b_ids - start_blk[e_of_b])[:, None] * BLOCK + jnp.arange(
        BLOCK, dtype=jnp.int32)[None, :]
    valid = p < counts[e_of_b][:, None]
    spos = jnp.clip(offsets[e_of_b][:, None] + p, 0, TOKENS - 1)
    tok = jnp.where(valid, sort_idx[spos], DUMP).astype(jnp.int32)
    wts = jnp.where(valid, top_k_weights[jnp.clip(tok, 0, TOKENS - 1), 0], 0.0)
    wts = wts.astype(jnp.float32).reshape(GRID, 1, BLOCK)

    grid_spec = pltpu.PrefetchScalarGridSpec(
        num_scalar_prefetch=3,
        grid=(GRID,),
        in_specs=[
            pl.BlockSpec((1, 1, BLOCK), lambda b, n, be, rid: (b, 0, 0)),
            pl.BlockSpec((TOKENS, HIDDEN), lambda b, n, be, rid: (0, 0)),
            pl.BlockSpec((1, 2 * FF, HIDDEN),
                         lambda b, n, be, rid: (be[b], 0, 0)),
            pl.BlockSpec((1, HIDDEN, FF),
                         lambda b, n, be, rid: (be[b], 0, 0)),
        ],
        out_specs=pl.BlockSpec((TOKENS + 8, HIDDEN),
                               lambda b, n, be, rid: (0, 0)),
        scratch_shapes=[
            pltpu.VMEM((BLOCK, HIDDEN), jnp.float32),
            pltpu.VMEM((BLOCK, HIDDEN), jnp.float32),
        ],
    )

    out = pl.pallas_call(
        _moe_kernel,
        grid_spec=grid_spec,
        out_shape=jax.ShapeDtypeStruct((TOKENS + 8, HIDDEN), jnp.float32),
    )(nblocks.reshape(1), e_of_b, tok.reshape(-1),
      wts, hidden_states, gate_up_proj, down_proj)
    return out[:TOKENS]


# speedup vs baseline: 1.4125x; 1.0307x over previous
import jax
import jax.numpy as jnp
from jax.experimental import pallas as pl
from jax.experimental.pallas import tpu as pltpu

NUM_EXPERTS = 64
HIDDEN = 768
FF = 512
TOKENS = 2048
BLOCK = 128
GRID = TOKENS // BLOCK + NUM_EXPERTS
NSLOTS = GRID * BLOCK
OUT_BLOCKS = TOKENS // BLOCK
DUMP = TOKENS


def _mlp_kernel(nblk_ref, bexp_ref,
                tok_ref, x_ref, gu_w_ref, dn_w_ref,
                y_ref,
                w1_s, w2_s):
    b = pl.program_id(0)

    changed = jnp.logical_or(b == 0, bexp_ref[b] != bexp_ref[jnp.maximum(b - 1, 0)])

    @pl.when(jnp.logical_and(b < nblk_ref[0], changed))
    def _():
        w1_s[...] = gu_w_ref[0].astype(jnp.bfloat16)
        w2_s[...] = dn_w_ref[0].astype(jnp.bfloat16)

    @pl.when(b < nblk_ref[0])
    def _():
        ids = tok_ref[0, 0, :]
        sel = (ids[:, None] == jax.lax.broadcasted_iota(
            jnp.int32, (BLOCK, TOKENS), 1))
        p = sel.astype(jnp.bfloat16)
        x = jax.lax.dot_general(
            p, x_ref[...], (((1,), (0,)), ((), ())),
            preferred_element_type=jnp.float32).astype(jnp.bfloat16)
        gu = jax.lax.dot_general(
            x, w1_s[...], (((1,), (1,)), ((), ())),
            preferred_element_type=jnp.float32)
        gate = gu[:, :FF]
        up = gu[:, FF:]
        h = (gate * jax.nn.sigmoid(gate) * up).astype(jnp.bfloat16)
        y = jax.lax.dot_general(
            h, w2_s[...], (((1,), (1,)), ((), ())),
            preferred_element_type=jnp.float32)
        y_ref[...] = y.astype(jnp.bfloat16)

    @pl.when(b >= nblk_ref[0])
    def _():
        y_ref[...] = jnp.zeros((BLOCK, HIDDEN), jnp.bfloat16)


def _combine_kernel(slot_ref, w_ref, y_ref, out_ref):
    slots = slot_ref[0, 0, :]
    w = w_ref[0, 0, :]
    sel = (slots[:, None] == jax.lax.broadcasted_iota(
        jnp.int32, (BLOCK, NSLOTS), 1))
    q = jnp.where(sel, w[:, None], 0.0).astype(jnp.bfloat16)
    out_ref[...] = jax.lax.dot_general(
        q, y_ref[...], (((1,), (0,)), ((), ())),
        preferred_element_type=jnp.float32)


@jax.jit
def kernel(hidden_states, top_k_index, top_k_weights, gate_up_proj, down_proj):
    e = top_k_index[:, 0].astype(jnp.int32)
    sort_idx = jnp.argsort(e).astype(jnp.int32)
    counts = jnp.bincount(e, length=NUM_EXPERTS).astype(jnp.int32)
    nb = (counts + BLOCK - 1) // BLOCK
    start_blk = jnp.cumsum(nb) - nb
    nblocks = jnp.sum(nb).astype(jnp.int32)
    offsets = jnp.cumsum(counts) - counts

    b_ids = jnp.arange(GRID, dtype=jnp.int32)
    e_of_b = (jnp.searchsorted(start_blk, b_ids, side="right") - 1).astype(
        jnp.int32)
    p = (b_ids - start_blk[e_of_b])[:, None] * BLOCK + jnp.arange(
        BLOCK, dtype=jnp.int32)[None, :]
    valid = p < counts[e_of_b][:, None]
    spos = jnp.clip(offsets[e_of_b][:, None] + p, 0, TOKENS - 1)
    tok = jnp.where(valid, sort_idx[spos], DUMP).astype(jnp.int32)

    flat_tok = tok.reshape(-1)
    slot_of = jnp.zeros((TOKENS,), jnp.int32).at[flat_tok].set(
        jnp.arange(NSLOTS, dtype=jnp.int32), mode="drop")

    y_sorted = pl.pallas_call(
        _mlp_kernel,
        grid_spec=pltpu.PrefetchScalarGridSpec(
            num_scalar_prefetch=2,
            grid=(GRID,),
            in_specs=[
                pl.BlockSpec((1, 1, BLOCK), lambda b, n, be: (b, 0, 0)),
                pl.BlockSpec((TOKENS, HIDDEN), lambda b, n, be: (0, 0)),
                pl.BlockSpec((1, 2 * FF, HIDDEN), lambda b, n, be: (be[b], 0, 0)),
                pl.BlockSpec((1, HIDDEN, FF), lambda b, n, be: (be[b], 0, 0)),
            ],
            out_specs=pl.BlockSpec((BLOCK, HIDDEN), lambda b, n, be: (b, 0)),
            scratch_shapes=[
                pltpu.VMEM((2 * FF, HIDDEN), jnp.bfloat16),
                pltpu.VMEM((HIDDEN, FF), jnp.bfloat16),
            ],
        ),
        out_shape=jax.ShapeDtypeStruct((NSLOTS, HIDDEN), jnp.bfloat16),
    )(nblocks.reshape(1), e_of_b, tok.reshape(GRID, 1, BLOCK),
      hidden_states.astype(jnp.bfloat16), gate_up_proj, down_proj)

    out = pl.pallas_call(
        _combine_kernel,
        grid=(OUT_BLOCKS,),
        in_specs=[
            pl.BlockSpec((1, 1, BLOCK), lambda b: (b, 0, 0)),
            pl.BlockSpec((1, 1, BLOCK), lambda b: (b, 0, 0)),
            pl.BlockSpec((NSLOTS, HIDDEN), lambda b: (0, 0)),
        ],
        out_specs=pl.BlockSpec((BLOCK, HIDDEN), lambda b: (b, 0)),
        out_shape=jax.ShapeDtypeStruct((TOKENS, HIDDEN), jnp.float32),
    )(slot_of.reshape(OUT_BLOCKS, 1, BLOCK),
      top_k_weights[:, 0].reshape(OUT_BLOCKS, 1, BLOCK).astype(jnp.float32),
      y_sorted)
    return out


# speedup vs baseline: 2.3297x; 1.6493x over previous
import numpy as np

import jax
import jax.numpy as jnp
from jax.experimental import pallas as pl
from jax.experimental.pallas import tpu as pltpu

NUM_EXPERTS = 64
HIDDEN = 768
FF = 512
TOKENS = 2048
BLOCK = 128
GRID = TOKENS // BLOCK + NUM_EXPERTS
NSLOTS = GRID * BLOCK
OUT_BLOCKS = TOKENS // BLOCK


def _routing_kernel(erow_ref, ecol_ref, u_ref, l_ref,
                    slotlane_ref, slotsub_ref, meta_ref):
    e_row = jnp.broadcast_to(erow_ref[0:1, :], (NUM_EXPERTS, TOKENS))
    onehot_t = (e_row == jax.lax.broadcasted_iota(
        jnp.int32, (NUM_EXPERTS, TOKENS), 0)).astype(jnp.bfloat16)
    onehot = (ecol_ref[:, 0:NUM_EXPERTS] == jax.lax.broadcasted_iota(
        jnp.int32, (TOKENS, NUM_EXPERTS), 1)).astype(jnp.bfloat16)

    counts = jax.lax.dot_general(
        jnp.ones((8, TOKENS), jnp.bfloat16), onehot_t,
        (((1,), (1,)), ((), ())),
        preferred_element_type=jnp.float32)
    nb = jnp.floor((counts + (BLOCK - 1)) * (1.0 / BLOCK))
    upper64 = (jax.lax.broadcasted_iota(
        jnp.int32, (NUM_EXPERTS, NUM_EXPERTS), 0)
        < jax.lax.broadcasted_iota(
            jnp.int32, (NUM_EXPERTS, NUM_EXPERTS), 1)).astype(jnp.bfloat16)
    start = jax.lax.dot_general(
        nb.astype(jnp.bfloat16), upper64, (((1,), (0,)), ((), ())),
        preferred_element_type=jnp.float32)
    nblocks = (start[0:1, NUM_EXPERTS - 1:] + nb[0:1, NUM_EXPERTS - 1:])

    prefix_row = jax.lax.dot_general(
        onehot_t, u_ref[...], (((1,), (0,)), ((), ())),
        preferred_element_type=jnp.float32)
    rank_row = jnp.sum(prefix_row * onehot_t.astype(jnp.float32),
                       axis=0, keepdims=True)
    slotbase_row = jax.lax.dot_general(
        start[0:1, :].astype(jnp.bfloat16), onehot_t,
        (((1,), (0,)), ((), ())),
        preferred_element_type=jnp.float32)
    slot_row = slotbase_row * float(BLOCK) + rank_row
    slotlane_ref[...] = jnp.broadcast_to(
        slot_row, (8, TOKENS)).astype(jnp.int32)

    prefix_col = jax.lax.dot_general(
        l_ref[...], onehot, (((1,), (0,)), ((), ())),
        preferred_element_type=jnp.float32)
    onehot_f = onehot.astype(jnp.float32)
    rank_col = jnp.sum(prefix_col * onehot_f,
                       axis=1, keepdims=True)
    start_bt = jnp.broadcast_to(start[0:1, :], (TOKENS, NUM_EXPERTS))
    slotbase_col = jnp.sum(start_bt * onehot_f,
                           axis=1, keepdims=True)
    slot_col = slotbase_col * float(BLOCK) + rank_col
    slotsub_ref[...] = jnp.broadcast_to(
        slot_col, (TOKENS, BLOCK)).astype(jnp.int32)

    start_bb = jnp.broadcast_to(start[0:1, :], (BLOCK, NUM_EXPERTS))
    b_sub = jax.lax.broadcasted_iota(
        jnp.int32, (BLOCK, NUM_EXPERTS), 0).astype(jnp.float32)
    e_of_b = jnp.sum((start_bb <= b_sub).astype(jnp.float32),
                     axis=1, keepdims=True) - 1.0
    lane = jax.lax.broadcasted_iota(jnp.int32, (BLOCK, 8), 1)
    meta = jnp.where(lane == 0, jnp.broadcast_to(e_of_b, (BLOCK, 8)),
                     jnp.broadcast_to(nblocks, (BLOCK, 8)))
    meta_ref[...] = meta.astype(jnp.int32)


def _mlp_kernel(meta_ref,
                slot_ref, x_ref, gu_w_ref, dn_w_ref,
                y_ref,
                w1_s, w2_s):
    b = pl.program_id(0)
    nblk = meta_ref[0, 1]

    changed = jnp.logical_or(
        b == 0, meta_ref[b, 0] != meta_ref[jnp.maximum(b - 1, 0), 0])

    @pl.when(jnp.logical_and(b < nblk, changed))
    def _():
        w1_s[...] = gu_w_ref[0].astype(jnp.bfloat16)
        w2_s[...] = dn_w_ref[0].astype(jnp.bfloat16)

    @pl.when(b < nblk)
    def _():
        slots = jnp.broadcast_to(slot_ref[0:1, :], (BLOCK, TOKENS))
        s_iota = jax.lax.broadcasted_iota(
            jnp.int32, (BLOCK, TOKENS), 0) + b * BLOCK
        p = (slots == s_iota).astype(jnp.bfloat16)
        x = jax.lax.dot_general(
            p, x_ref[...], (((1,), (0,)), ((), ())),
            preferred_element_type=jnp.float32).astype(jnp.bfloat16)
        gu = jax.lax.dot_general(
            x, w1_s[...], (((1,), (1,)), ((), ())),
            preferred_element_type=jnp.float32)
        gate = gu[:, :FF]
        up = gu[:, FF:]
        h = (gate * jax.nn.sigmoid(gate) * up).astype(jnp.bfloat16)
        y = jax.lax.dot_general(
            h, w2_s[...], (((1,), (1,)), ((), ())),
            preferred_element_type=jnp.float32)
        y_ref[...] = y.astype(jnp.bfloat16)

    @pl.when(b >= nblk)
    def _():
        y_ref[...] = jnp.zeros((BLOCK, HIDDEN), jnp.bfloat16)


def _combine_kernel(slot_ref, w_ref, y_ref, out_ref):
    slot_sub = jnp.broadcast_to(slot_ref[:, 0:1], (BLOCK, NSLOTS))
    sel = slot_sub == jax.lax.broadcasted_iota(
        jnp.int32, (BLOCK, NSLOTS), 1)
    w_sub = jnp.broadcast_to(w_ref[:, 0:1], (BLOCK, NSLOTS))
    q = jnp.where(sel, w_sub, 0.0).astype(jnp.bfloat16)
    out_ref[...] = jax.lax.dot_general(
        q, y_ref[...], (((1,), (0,)), ((), ())),
        preferred_element_type=jnp.float32)


def _tri_consts():
    upper = np.triu(np.ones((TOKENS, TOKENS), np.float32), 1)
    return (jnp.asarray(upper, dtype=jnp.bfloat16),
            jnp.asarray(upper.T, dtype=jnp.bfloat16))


@jax.jit
def kernel(hidden_states, top_k_index, top_k_weights, gate_up_proj, down_proj):
    e32 = top_k_index.astype(jnp.int32)
    e_row = jnp.broadcast_to(e32.reshape(1, TOKENS), (8, TOKENS))
    e_col = jnp.broadcast_to(e32.reshape(TOKENS, 1), (TOKENS, BLOCK))
    u_strict, l_strict = _tri_consts()

    slot_lane, slot_sub, meta = pl.pallas_call(
        _routing_kernel,
        grid=(1,),
        in_specs=[
            pl.BlockSpec((8, TOKENS), lambda i: (0, 0)),
            pl.BlockSpec((TOKENS, BLOCK), lambda i: (0, 0)),
            pl.BlockSpec((TOKENS, TOKENS), lambda i: (0, 0)),
            pl.BlockSpec((TOKENS, TOKENS), lambda i: (0, 0)),
        ],
        out_specs=[
            pl.BlockSpec((8, TOKENS), lambda i: (0, 0)),
            pl.BlockSpec((TOKENS, BLOCK), lambda i: (0, 0)),
            pl.BlockSpec((BLOCK, 8), lambda i: (0, 0)),
        ],
        out_shape=[
            jax.ShapeDtypeStruct((8, TOKENS), jnp.int32),
            jax.ShapeDtypeStruct((TOKENS, BLOCK), jnp.int32),
            jax.ShapeDtypeStruct((BLOCK, 8), jnp.int32),
        ],
    )(e_row, e_col, u_strict, l_strict)

    y_sorted = pl.pallas_call(
        _mlp_kernel,
        grid_spec=pltpu.PrefetchScalarGridSpec(
            num_scalar_prefetch=1,
            grid=(GRID,),
            in_specs=[
                pl.BlockSpec((8, TOKENS), lambda b, m: (0, 0)),
                pl.BlockSpec((TOKENS, HIDDEN), lambda b, m: (0, 0)),
                pl.BlockSpec((1, 2 * FF, HIDDEN), lambda b, m: (m[b, 0], 0, 0)),
                pl.BlockSpec((1, HIDDEN, FF), lambda b, m: (m[b, 0], 0, 0)),
            ],
            out_specs=pl.BlockSpec((BLOCK, HIDDEN), lambda b, m: (b, 0)),
            scratch_shapes=[
                pltpu.VMEM((2 * FF, HIDDEN), jnp.bfloat16),
                pltpu.VMEM((HIDDEN, FF), jnp.bfloat16),
            ],
        ),
        out_shape=jax.ShapeDtypeStruct((NSLOTS, HIDDEN), jnp.bfloat16),
    )(meta, slot_lane, hidden_states.astype(jnp.bfloat16),
      gate_up_proj, down_proj)

    w_col = jnp.broadcast_to(
        top_k_weights.astype(jnp.float32).reshape(TOKENS, 1), (TOKENS, BLOCK))
    out = pl.pallas_call(
        _combine_kernel,
        grid=(OUT_BLOCKS,),
        in_specs=[
            pl.BlockSpec((BLOCK, BLOCK), lambda b: (b, 0)),
            pl.BlockSpec((BLOCK, BLOCK), lambda b: (b, 0)),
            pl.BlockSpec((NSLOTS, HIDDEN), lambda b: (0, 0)),
        ],
        out_specs=pl.BlockSpec((BLOCK, HIDDEN), lambda b: (b, 0)),
        out_shape=jax.ShapeDtypeStruct((TOKENS, HIDDEN), jnp.float32),
    )(slot_sub, w_col, y_sorted)
    return out


# speedup vs baseline: 2.3438x; 1.0061x over previous
import numpy as np

import jax
import jax.numpy as jnp
from jax.experimental import pallas as pl
from jax.experimental.pallas import tpu as pltpu

NUM_EXPERTS = 64
HIDDEN = 768
FF = 512
TOKENS = 2048
BLOCK = 128
GRID = TOKENS // BLOCK + NUM_EXPERTS
NSLOTS = GRID * BLOCK
OUT_BLOCKS = TOKENS // BLOCK


def _routing_kernel(erow_ref, ecol_ref, u_ref, l_ref, x_ref,
                    slotlane_ref, slotsub_ref, meta_ref, xbf_ref):
    xbf_ref[...] = x_ref[...].astype(jnp.bfloat16)

    e_row = jnp.broadcast_to(erow_ref[0:1, :], (NUM_EXPERTS, TOKENS))
    onehot_t = (e_row == jax.lax.broadcasted_iota(
        jnp.int32, (NUM_EXPERTS, TOKENS), 0)).astype(jnp.bfloat16)
    e_col = jnp.broadcast_to(ecol_ref[:, 0:1], (TOKENS, NUM_EXPERTS))
    onehot = (e_col == jax.lax.broadcasted_iota(
        jnp.int32, (TOKENS, NUM_EXPERTS), 1)).astype(jnp.bfloat16)

    counts = jax.lax.dot_general(
        jnp.ones((8, TOKENS), jnp.bfloat16), onehot_t,
        (((1,), (1,)), ((), ())),
        preferred_element_type=jnp.float32)
    nb = jnp.floor((counts + (BLOCK - 1)) * (1.0 / BLOCK))
    upper64 = (jax.lax.broadcasted_iota(
        jnp.int32, (NUM_EXPERTS, NUM_EXPERTS), 0)
        < jax.lax.broadcasted_iota(
            jnp.int32, (NUM_EXPERTS, NUM_EXPERTS), 1)).astype(jnp.bfloat16)
    start = jax.lax.dot_general(
        nb.astype(jnp.bfloat16), upper64, (((1,), (0,)), ((), ())),
        preferred_element_type=jnp.float32)
    nblocks = (start[0:1, NUM_EXPERTS - 1:] + nb[0:1, NUM_EXPERTS - 1:])

    prefix_row = jax.lax.dot_general(
        onehot_t, u_ref[...], (((1,), (0,)), ((), ())),
        preferred_element_type=jnp.float32)
    rank_row = jnp.sum(prefix_row * onehot_t.astype(jnp.float32),
                       axis=0, keepdims=True)
    slotbase_row = jax.lax.dot_general(
        start[0:1, :].astype(jnp.bfloat16), onehot_t,
        (((1,), (0,)), ((), ())),
        preferred_element_type=jnp.float32)
    slot_row = slotbase_row * float(BLOCK) + rank_row
    slotlane_ref[...] = jnp.broadcast_to(
        slot_row, (8, TOKENS)).astype(jnp.int32)

    prefix_col = jax.lax.dot_general(
        l_ref[...], onehot, (((1,), (0,)), ((), ())),
        preferred_element_type=jnp.float32)
    onehot_f = onehot.astype(jnp.float32)
    rank_col = jnp.sum(prefix_col * onehot_f,
                       axis=1, keepdims=True)
    start_bt = jnp.broadcast_to(start[0:1, :], (TOKENS, NUM_EXPERTS))
    slotbase_col = jnp.sum(start_bt * onehot_f,
                           axis=1, keepdims=True)
    slot_col = slotbase_col * float(BLOCK) + rank_col
    slotsub_ref[...] = jnp.broadcast_to(
        slot_col, (TOKENS, 8)).astype(jnp.int32)

    start_bb = jnp.broadcast_to(start[0:1, :], (BLOCK, NUM_EXPERTS))
    b_sub = jax.lax.broadcasted_iota(
        jnp.int32, (BLOCK, NUM_EXPERTS), 0).astype(jnp.float32)
    e_of_b = jnp.sum((start_bb <= b_sub).astype(jnp.float32),
                     axis=1, keepdims=True) - 1.0
    lane = jax.lax.broadcasted_iota(jnp.int32, (BLOCK, 8), 1)
    meta = jnp.where(lane == 0, jnp.broadcast_to(e_of_b, (BLOCK, 8)),
                     jnp.broadcast_to(nblocks, (BLOCK, 8)))
    meta_ref[...] = meta.astype(jnp.int32)


def _mlp_kernel(meta_ref,
                slot_ref, x_ref, gu_w_ref, dn_w_ref,
                y_ref,
                w1_s, w2_s):
    b = pl.program_id(0)
    nblk = meta_ref[0, 1]

    changed = jnp.logical_or(
        b == 0, meta_ref[b, 0] != meta_ref[jnp.maximum(b - 1, 0), 0])

    @pl.when(jnp.logical_and(b < nblk, changed))
    def _():
        w1_s[...] = gu_w_ref[0].astype(jnp.bfloat16)
        w2_s[...] = dn_w_ref[0].astype(jnp.bfloat16)

    @pl.when(b < nblk)
    def _():
        slots = jnp.broadcast_to(slot_ref[0:1, :], (BLOCK, TOKENS))
        s_iota = jax.lax.broadcasted_iota(
            jnp.int32, (BLOCK, TOKENS), 0) + b * BLOCK
        p = (slots == s_iota).astype(jnp.bfloat16)
        x = jax.lax.dot_general(
            p, x_ref[...], (((1,), (0,)), ((), ())),
            preferred_element_type=jnp.float32).astype(jnp.bfloat16)
        gu = jax.lax.dot_general(
            x, w1_s[...], (((1,), (1,)), ((), ())),
            preferred_element_type=jnp.float32)
        gate = gu[:, :FF]
        up = gu[:, FF:]
        h = (gate * jax.nn.sigmoid(gate) * up).astype(jnp.bfloat16)
        y = jax.lax.dot_general(
            h, w2_s[...], (((1,), (1,)), ((), ())),
            preferred_element_type=jnp.float32)
        y_ref[...] = y.astype(jnp.bfloat16)

    @pl.when(b >= nblk)
    def _():
        y_ref[...] = jnp.zeros((BLOCK, HIDDEN), jnp.bfloat16)


def _combine_kernel(slot_ref, w_ref, y_ref, out_ref):
    slot_sub = jnp.broadcast_to(slot_ref[:, 0:1], (BLOCK, NSLOTS))
    sel = slot_sub == jax.lax.broadcasted_iota(
        jnp.int32, (BLOCK, NSLOTS), 1)
    w_sub = jnp.broadcast_to(w_ref[:, 0:1], (BLOCK, NSLOTS))
    q = jnp.where(sel, w_sub, 0.0).astype(jnp.bfloat16)
    out_ref[...] = jax.lax.dot_general(
        q, y_ref[...], (((1,), (0,)), ((), ())),
        preferred_element_type=jnp.float32)


def _tri_consts():
    upper = np.triu(np.ones((TOKENS, TOKENS), np.float32), 1)
    return (jnp.asarray(upper, dtype=jnp.bfloat16),
            jnp.asarray(upper.T, dtype=jnp.bfloat16))


@jax.jit
def kernel(hidden_states, top_k_index, top_k_weights, gate_up_proj, down_proj):
    e32 = top_k_index.astype(jnp.int32)
    e_row = jnp.broadcast_to(e32.reshape(1, TOKENS), (8, TOKENS))
    e_col = jnp.broadcast_to(e32.reshape(TOKENS, 1), (TOKENS, 8))
    u_strict, l_strict = _tri_consts()

    slot_lane, slot_sub, meta, x_bf = pl.pallas_call(
        _routing_kernel,
        grid=(1,),
        in_specs=[
            pl.BlockSpec((8, TOKENS), lambda i: (0, 0)),
            pl.BlockSpec((TOKENS, 8), lambda i: (0, 0)),
            pl.BlockSpec((TOKENS, TOKENS), lambda i: (0, 0)),
            pl.BlockSpec((TOKENS, TOKENS), lambda i: (0, 0)),
            pl.BlockSpec((TOKENS, HIDDEN), lambda i: (0, 0)),
        ],
        out_specs=[
            pl.BlockSpec((8, TOKENS), lambda i: (0, 0)),
            pl.BlockSpec((TOKENS, 8), lambda i: (0, 0)),
            pl.BlockSpec((BLOCK, 8), lambda i: (0, 0)),
            pl.BlockSpec((TOKENS, HIDDEN), lambda i: (0, 0)),
        ],
        out_shape=[
            jax.ShapeDtypeStruct((8, TOKENS), jnp.int32),
            jax.ShapeDtypeStruct((TOKENS, 8), jnp.int32),
            jax.ShapeDtypeStruct((BLOCK, 8), jnp.int32),
            jax.ShapeDtypeStruct((TOKENS, HIDDEN), jnp.bfloat16),
        ],
    )(e_row, e_col, u_strict, l_strict, hidden_states)

    y_sorted = pl.pallas_call(
        _mlp_kernel,
        grid_spec=pltpu.PrefetchScalarGridSpec(
            num_scalar_prefetch=1,
            grid=(GRID,),
            in_specs=[
                pl.BlockSpec((8, TOKENS), lambda b, m: (0, 0)),
                pl.BlockSpec((TOKENS, HIDDEN), lambda b, m: (0, 0)),
                pl.BlockSpec((1, 2 * FF, HIDDEN), lambda b, m: (m[b, 0], 0, 0)),
                pl.BlockSpec((1, HIDDEN, FF), lambda b, m: (m[b, 0], 0, 0)),
            ],
            out_specs=pl.BlockSpec((BLOCK, HIDDEN), lambda b, m: (b, 0)),
            scratch_shapes=[
                pltpu.VMEM((2 * FF, HIDDEN), jnp.bfloat16),
                pltpu.VMEM((HIDDEN, FF), jnp.bfloat16),
            ],
        ),
        out_shape=jax.ShapeDtypeStruct((NSLOTS, HIDDEN), jnp.bfloat16),
    )(meta, slot_lane, x_bf, gate_up_proj, down_proj)

    w_col = jnp.broadcast_to(
        top_k_weights.astype(jnp.float32).reshape(TOKENS, 1), (TOKENS, 8))
    out = pl.pallas_call(
        _combine_kernel,
        grid=(OUT_BLOCKS,),
        in_specs=[
            pl.BlockSpec((BLOCK, 8), lambda b: (b, 0)),
            pl.BlockSpec((BLOCK, 8), lambda b: (b, 0)),
            pl.BlockSpec((NSLOTS, HIDDEN), lambda b: (0, 0)),
        ],
        out_specs=pl.BlockSpec((BLOCK, HIDDEN), lambda b: (b, 0)),
        out_shape=jax.ShapeDtypeStruct((TOKENS, HIDDEN), jnp.float32),
    )(slot_sub, w_col, y_sorted)
    return out


# speedup vs baseline: 2.4256x; 1.0349x over previous
import numpy as np

import jax
import jax.numpy as jnp
from jax.experimental import pallas as pl
from jax.experimental.pallas import tpu as pltpu

NUM_EXPERTS = 64
HIDDEN = 768
FF = 512
TOKENS = 2048
BLOCK = 128
GRID = TOKENS // BLOCK + NUM_EXPERTS
NSLOTS = GRID * BLOCK
OUT_BLOCKS = TOKENS // BLOCK


def _routing_kernel(erow_ref, ecol_ref, u_ref, l_ref, x_ref,
                    slotlane_ref, slotsub_ref, meta_ref, xbf_ref):
    xbf_ref[...] = x_ref[...].astype(jnp.bfloat16)

    e_row = jnp.broadcast_to(erow_ref[0:1, :], (NUM_EXPERTS, TOKENS))
    onehot_t = (e_row == jax.lax.broadcasted_iota(
        jnp.int32, (NUM_EXPERTS, TOKENS), 0)).astype(jnp.bfloat16)
    e_col = jnp.broadcast_to(ecol_ref[:, 0:1], (TOKENS, NUM_EXPERTS))
    onehot = (e_col == jax.lax.broadcasted_iota(
        jnp.int32, (TOKENS, NUM_EXPERTS), 1)).astype(jnp.bfloat16)

    counts = jax.lax.dot_general(
        jnp.ones((8, TOKENS), jnp.bfloat16), onehot_t,
        (((1,), (1,)), ((), ())),
        preferred_element_type=jnp.float32)
    nb = jnp.floor((counts + (BLOCK - 1)) * (1.0 / BLOCK))
    upper64 = (jax.lax.broadcasted_iota(
        jnp.int32, (NUM_EXPERTS, NUM_EXPERTS), 0)
        < jax.lax.broadcasted_iota(
            jnp.int32, (NUM_EXPERTS, NUM_EXPERTS), 1)).astype(jnp.bfloat16)
    start = jax.lax.dot_general(
        nb.astype(jnp.bfloat16), upper64, (((1,), (0,)), ((), ())),
        preferred_element_type=jnp.float32)
    nblocks = (start[0:1, NUM_EXPERTS - 1:] + nb[0:1, NUM_EXPERTS - 1:])

    prefix_row = jax.lax.dot_general(
        onehot_t, u_ref[...], (((1,), (0,)), ((), ())),
        preferred_element_type=jnp.float32)
    rank_row = jnp.sum(prefix_row * onehot_t.astype(jnp.float32),
                       axis=0, keepdims=True)
    slotbase_row = jax.lax.dot_general(
        start[0:1, :].astype(jnp.bfloat16), onehot_t,
        (((1,), (0,)), ((), ())),
        preferred_element_type=jnp.float32)
    slot_row = slotbase_row * float(BLOCK) + rank_row
    slotlane_ref[...] = jnp.broadcast_to(
        slot_row, (8, TOKENS)).astype(jnp.int32)

    prefix_col = jax.lax.dot_general(
        l_ref[...], onehot, (((1,), (0,)), ((), ())),
        preferred_element_type=jnp.float32)
    onehot_f = onehot.astype(jnp.float32)
    rank_col = jnp.sum(prefix_col * onehot_f,
                       axis=1, keepdims=True)
    start_bt = jnp.broadcast_to(start[0:1, :], (TOKENS, NUM_EXPERTS))
    slotbase_col = jnp.sum(start_bt * onehot_f,
                           axis=1, keepdims=True)
    slot_col = slotbase_col * float(BLOCK) + rank_col
    slotsub_ref[...] = jnp.broadcast_to(
        slot_col, (TOKENS, 8)).astype(jnp.int32)

    start_bb = jnp.broadcast_to(start[0:1, :], (BLOCK, NUM_EXPERTS))
    b_sub = jax.lax.broadcasted_iota(
        jnp.int32, (BLOCK, NUM_EXPERTS), 0).astype(jnp.float32)
    e_of_b = jnp.sum((start_bb <= b_sub).astype(jnp.float32),
                     axis=1, keepdims=True) - 1.0
    lane = jax.lax.broadcasted_iota(jnp.int32, (BLOCK, 8), 1)
    meta = jnp.where(lane == 0, jnp.broadcast_to(e_of_b, (BLOCK, 8)),
                     jnp.broadcast_to(nblocks, (BLOCK, 8)))
    meta_ref[...] = meta.astype(jnp.int32)


def _mlp_kernel(meta_ref,
                slot_ref, x_ref, gu_w_ref, dn_w_ref,
                y_ref,
                w1_s, w2_s):
    b = pl.program_id(0)
    nblk = meta_ref[0, 1]

    @pl.when(b < nblk)
    def _():
        w1_s[...] = gu_w_ref[0].astype(jnp.bfloat16)
        w2_s[...] = dn_w_ref[0].astype(jnp.bfloat16)

    @pl.when(b < nblk)
    def _():
        slots = jnp.broadcast_to(slot_ref[0:1, :], (BLOCK, TOKENS))
        s_iota = jax.lax.broadcasted_iota(
            jnp.int32, (BLOCK, TOKENS), 0) + b * BLOCK
        p = (slots == s_iota).astype(jnp.bfloat16)
        x = jax.lax.dot_general(
            p, x_ref[...], (((1,), (0,)), ((), ())),
            preferred_element_type=jnp.float32).astype(jnp.bfloat16)
        gu = jax.lax.dot_general(
            x, w1_s[...], (((1,), (1,)), ((), ())),
            preferred_element_type=jnp.float32)
        gate = gu[:, :FF]
        up = gu[:, FF:]
        h = (gate * jax.nn.sigmoid(gate) * up).astype(jnp.bfloat16)
        y = jax.lax.dot_general(
            h, w2_s[...], (((1,), (1,)), ((), ())),
            preferred_element_type=jnp.float32)
        y_ref[...] = y.astype(jnp.bfloat16)

    @pl.when(b >= nblk)
    def _():
        y_ref[...] = jnp.zeros((BLOCK, HIDDEN), jnp.bfloat16)


def _combine_kernel(slot_ref, w_ref, y_ref, out_ref):
    slot_sub = jnp.broadcast_to(slot_ref[:, 0:1], (BLOCK, NSLOTS))
    sel = slot_sub == jax.lax.broadcasted_iota(
        jnp.int32, (BLOCK, NSLOTS), 1)
    w_sub = jnp.broadcast_to(w_ref[:, 0:1], (BLOCK, NSLOTS))
    q = jnp.where(sel, w_sub, 0.0).astype(jnp.bfloat16)
    out_ref[...] = jax.lax.dot_general(
        q, y_ref[...], (((1,), (0,)), ((), ())),
        preferred_element_type=jnp.float32)


def _tri_consts():
    upper = np.triu(np.ones((TOKENS, TOKENS), np.float32), 1)
    return (jnp.asarray(upper, dtype=jnp.bfloat16),
            jnp.asarray(upper.T, dtype=jnp.bfloat16))


@jax.jit
def kernel(hidden_states, top_k_index, top_k_weights, gate_up_proj, down_proj):
    e32 = top_k_index.astype(jnp.int32)
    e_row = jnp.broadcast_to(e32.reshape(1, TOKENS), (8, TOKENS))
    e_col = jnp.broadcast_to(e32.reshape(TOKENS, 1), (TOKENS, 8))
    u_strict, l_strict = _tri_consts()

    slot_lane, slot_sub, meta, x_bf = pl.pallas_call(
        _routing_kernel,
        grid=(1,),
        in_specs=[
            pl.BlockSpec((8, TOKENS), lambda i: (0, 0)),
            pl.BlockSpec((TOKENS, 8), lambda i: (0, 0)),
            pl.BlockSpec((TOKENS, TOKENS), lambda i: (0, 0)),
            pl.BlockSpec((TOKENS, TOKENS), lambda i: (0, 0)),
            pl.BlockSpec((TOKENS, HIDDEN), lambda i: (0, 0)),
        ],
        out_specs=[
            pl.BlockSpec((8, TOKENS), lambda i: (0, 0)),
            pl.BlockSpec((TOKENS, 8), lambda i: (0, 0)),
            pl.BlockSpec((BLOCK, 8), lambda i: (0, 0)),
            pl.BlockSpec((TOKENS, HIDDEN), lambda i: (0, 0)),
        ],
        out_shape=[
            jax.ShapeDtypeStruct((8, TOKENS), jnp.int32),
            jax.ShapeDtypeStruct((TOKENS, 8), jnp.int32),
            jax.ShapeDtypeStruct((BLOCK, 8), jnp.int32),
            jax.ShapeDtypeStruct((TOKENS, HIDDEN), jnp.bfloat16),
        ],
    )(e_row, e_col, u_strict, l_strict, hidden_states)

    y_sorted = pl.pallas_call(
        _mlp_kernel,
        grid_spec=pltpu.PrefetchScalarGridSpec(
            num_scalar_prefetch=1,
            grid=(GRID,),
            in_specs=[
                pl.BlockSpec((8, TOKENS), lambda b, m: (0, 0)),
                pl.BlockSpec((TOKENS, HIDDEN), lambda b, m: (0, 0)),
                pl.BlockSpec((1, 2 * FF, HIDDEN), lambda b, m: (m[b, 0], 0, 0)),
                pl.BlockSpec((1, HIDDEN, FF), lambda b, m: (m[b, 0], 0, 0)),
            ],
            out_specs=pl.BlockSpec((BLOCK, HIDDEN), lambda b, m: (b, 0)),
            scratch_shapes=[
                pltpu.VMEM((2 * FF, HIDDEN), jnp.bfloat16),
                pltpu.VMEM((HIDDEN, FF), jnp.bfloat16),
            ],
        ),
        compiler_params=pltpu.CompilerParams(
            dimension_semantics=("parallel",)),
        out_shape=jax.ShapeDtypeStruct((NSLOTS, HIDDEN), jnp.bfloat16),
    )(meta, slot_lane, x_bf, gate_up_proj, down_proj)

    w_col = jnp.broadcast_to(
        top_k_weights.astype(jnp.float32).reshape(TOKENS, 1), (TOKENS, 8))
    out = pl.pallas_call(
        _combine_kernel,
        grid=(OUT_BLOCKS,),
        in_specs=[
            pl.BlockSpec((BLOCK, 8), lambda b: (b, 0)),
            pl.BlockSpec((BLOCK, 8), lambda b: (b, 0)),
            pl.BlockSpec((NSLOTS, HIDDEN), lambda b: (0, 0)),
        ],
        out_specs=pl.BlockSpec((BLOCK, HIDDEN), lambda b: (b, 0)),
        compiler_params=pltpu.CompilerParams(
            dimension_semantics=("parallel",)),
        out_shape=jax.ShapeDtypeStruct((TOKENS, HIDDEN), jnp.float32),
    )(slot_sub, w_col, y_sorted)
    return out
